# baseline (speedup 1.0000x reference)
import functools

import jax
import jax.numpy as jnp
from jax import lax
from jax.experimental import pallas as pl
from jax.experimental.pallas import tpu as pltpu
from jax.experimental.pallas import tpu_sc as plsc

D = 64
LANE = 128
K = 4
CHUNK = K * LANE
VB = 4096
NPAIR = 123 * VB
HALF1_BASE = 122 * VB


def _conv1_body(x1_ref, x2_ref, o_ref):
    o_ref[:, 0:D] = x1_ref[...].T
    o_ref[:, D : 2 * D] = x2_ref[...].T


def _repack_table(tT):
    return pl.pallas_call(
        _conv1_body,
        grid=(NPAIR // VB,),
        in_specs=[
            pl.BlockSpec((D, VB), lambda i: (0, i)),
            pl.BlockSpec((D, VB), lambda i: (0, HALF1_BASE // VB + i)),
        ],
        out_specs=pl.BlockSpec((VB, 128), lambda i: (i, 0)),
        out_shape=jax.ShapeDtypeStruct((NPAIR, 128), jnp.float32),
    )(tT, tT)


@functools.partial(jax.jit, static_argnums=(0, 1))
def _gather_call(BT, S, idx2d, table128):
    B = BT * S
    info = plsc.get_sparse_core_info()
    nw = info.num_cores * info.num_subcores
    assert B % (nw * CHUNK) == 0 and BT % CHUNK == 0
    n_chunks = B // (nw * CHUNK)
    mesh = plsc.VectorSubcoreMesh(core_axis_name="c", subcore_axis_name="s")

    @functools.partial(
        pl.kernel,
        mesh=mesh,
        out_type=jax.ShapeDtypeStruct((S, D, BT), jnp.float32),
        scratch_types=[
            pltpu.VMEM((K, LANE), jnp.int32),
            pltpu.VMEM((CHUNK,), jnp.int32),
            pltpu.VMEM((CHUNK, 128), jnp.float32),
            pltpu.VMEM((D, CHUNK), jnp.float32),
            pltpu.SemaphoreType.DMA,
        ],
        compiler_params=pltpu.CompilerParams(
            use_tc_tiling_on_sc=False, needs_layout_passes=False
        ),
    )
    def gather_kernel(
        idx_hbm, table_hbm, out_hbm, idx_v, hcol_v, rows_v, t_v, sem
    ):
        wid = lax.axis_index("s") * info.num_cores + lax.axis_index("c")
        r_base = wid * (n_chunks * CHUNK)
        lanes = lax.iota(jnp.int32, 16)

        def chunk_body(i, carry):
            r0 = r_base + i * CHUNK
            pltpu.sync_copy(idx_hbm.at[pl.ds(r0 // LANE, K)], idx_v)
            for j in range(K):
                for k in range(LANE // 16):
                    v = idx_v[j, pl.ds(k * 16, 16)]
                    in0 = v < NPAIR
                    idx_v[j, pl.ds(k * 16, 16)] = jnp.where(
                        in0, v, v - HALF1_BASE
                    )
                    hcol_v[pl.ds(j * LANE + k * 16, 16)] = jnp.where(
                        in0, 0, D
                    )
            cps = [
                pltpu.async_copy(
                    table_hbm.at[idx_v.at[j]],
                    rows_v.at[pl.ds(j * LANE, LANE)],
                    sem,
                )
                for j in range(K)
            ]
            for cp in cps:
                cp.wait()

            def tbody(g, carry2):
                rowv = g * 16 + lanes
                colb = hcol_v[pl.ds(g * 16, 16)]
                for d in range(D):
                    vals = plsc.load_gather(rows_v, [rowv, colb + d])
                    t_v[d, pl.ds(g * 16, 16)] = vals
                return carry2

            lax.fori_loop(0, CHUNK // 16, tbody, 0)

            s_slot = r0 // BT
            b0 = r0 % BT
            pltpu.sync_copy(t_v, out_hbm.at[s_slot, :, pl.ds(b0, CHUNK)])
            return carry

        lax.fori_loop(0, n_chunks, chunk_body, 0)

    return gather_kernel(idx2d, table128)


def kernel(tokens, tok_embeddings):
    bt, s = tokens.shape
    B = bt * s
    table128 = _repack_table(tok_embeddings.T.astype(jnp.float32))
    idx2d = tokens.T.astype(jnp.int32).reshape(B // LANE, LANE)
    res = _gather_call(bt, s, idx2d, table128)
    return res.transpose(2, 0, 1)
